# Initial kernel scaffold; baseline (speedup 1.0000x reference)
import jax
import jax.numpy as jnp
from jax.experimental import pallas as pl


def kernel(x, pos, batch, sa1_mlp, sa2_mlp, sa3_mlp, head_mlp):
    raise NotImplementedError("write your pallas kernel here")


# speedup vs baseline: 9.9831x; 9.9831x over previous
import functools

import jax
import jax.numpy as jnp
import numpy as np
from jax import lax
from jax.experimental import pallas as pl
from jax.experimental.pallas import tpu as pltpu
from jax.experimental.pallas import tpu_sc as plsc

_B, _N, _F = 16, 1024, 3
_NCLS = 40
_SA1 = (512, 32, 0.2)
_SA2 = (128, 32, 0.4)
_EPS = 1e-5
_NEG = float("-inf")


def _fps_body(pos_ref, posq_ref):
    _, b, n = pos_ref.shape
    m = posq_ref.shape[2]
    px = pos_ref[0]
    py = pos_ref[1]
    pz = pos_ref[2]
    iota_n = lax.broadcasted_iota(jnp.int32, (b, n), 1)
    iota_m = lax.broadcasted_iota(jnp.int32, (b, m), 1)

    def gather_last(prev):
        eq = iota_n == prev[:, None]
        lx = jnp.sum(jnp.where(eq, px, 0.0), axis=1)
        ly = jnp.sum(jnp.where(eq, py, 0.0), axis=1)
        lz = jnp.sum(jnp.where(eq, pz, 0.0), axis=1)
        return lx, ly, lz

    def body(i, carry):
        prev, dists, qx, qy, qz = carry
        lx, ly, lz = gather_last(prev)
        hit = iota_m == (i - 1)
        qx = jnp.where(hit, lx[:, None], qx)
        qy = jnp.where(hit, ly[:, None], qy)
        qz = jnp.where(hit, lz[:, None], qz)
        dx = px - lx[:, None]
        dy = py - ly[:, None]
        dz = pz - lz[:, None]
        d = (dx * dx + dy * dy) + dz * dz
        dists = jnp.minimum(dists, d)
        am = jnp.argmax(dists, axis=1).astype(jnp.int32)
        return am, dists, qx, qy, qz

    prev0 = jnp.zeros((b,), jnp.int32)
    d0 = jnp.full((b, n), jnp.inf, jnp.float32)
    q0 = jnp.zeros((b, m), jnp.float32)
    prev, _, qx, qy, qz = lax.fori_loop(1, m, body, (prev0, d0, q0, q0, q0))
    lx, ly, lz = gather_last(prev)
    hit = iota_m == (m - 1)
    posq_ref[0] = jnp.where(hit, lx[:, None], qx)
    posq_ref[1] = jnp.where(hit, ly[:, None], qy)
    posq_ref[2] = jnp.where(hit, lz[:, None], qz)


def _fps(pos_t, m):
    _, b, n = pos_t.shape
    return pl.pallas_call(
        _fps_body,
        out_shape=jax.ShapeDtypeStruct((3, b, m), jnp.float32),
    )(pos_t)


def _knn_body(n, k, r2, posa_ref, posq_ref, idx_ref, bias_ref):
    m = posq_ref.shape[2]
    bidx = pl.program_id(0)
    qx = posq_ref[0, 0, :]
    qy = posq_ref[0, 1, :]
    qz = posq_ref[0, 2, :]
    px = posa_ref[0, 0, :]
    py = posa_ref[0, 1, :]
    pz = posa_ref[0, 2, :]
    dx = qx[:, None] - px[None, :]
    dy = qy[:, None] - py[None, :]
    dz = qz[:, None] - pz[None, :]
    d2 = (dx * dx + dy * dy) + dz * dz
    iota_n = lax.broadcasted_iota(jnp.int32, (m, n), 1)
    iota_k = lax.broadcasted_iota(jnp.int32, (m, k), 1)
    base = bidx * n

    def body(t, carry):
        d2c, idxc, biasc = carry
        cur = jnp.min(d2c, axis=1)
        am = jnp.argmin(d2c, axis=1).astype(jnp.int32)
        hit = iota_k == t
        idxc = jnp.where(hit, am[:, None] + base, idxc)
        ok = jnp.where(cur <= r2, 0.0, _NEG)
        biasc = jnp.where(hit, ok[:, None], biasc)
        d2c = jnp.where(iota_n == am[:, None], jnp.inf, d2c)
        return d2c, idxc, biasc

    idx0 = jnp.zeros((m, k), jnp.int32)
    bias0 = jnp.zeros((m, k), jnp.float32)
    _, idxc, biasc = lax.fori_loop(0, k, body, (d2, idx0, bias0))
    idx_ref[0] = idxc
    bias_ref[0] = biasc


def _knn(posa_t, posq_t, k, r):
    b, _, n = posa_t.shape
    m = posq_t.shape[2]
    r2 = float(r) * float(r)
    return pl.pallas_call(
        functools.partial(_knn_body, n, k, r2),
        grid=(b,),
        in_specs=[
            pl.BlockSpec((1, 3, n), lambda i: (i, 0, 0)),
            pl.BlockSpec((1, 3, m), lambda i: (i, 0, 0)),
        ],
        out_specs=[
            pl.BlockSpec((1, m, k), lambda i: (i, 0, 0)),
            pl.BlockSpec((1, m, k), lambda i: (i, 0, 0)),
        ],
        out_shape=[
            jax.ShapeDtypeStruct((b, m, k), jnp.int32),
            jax.ShapeDtypeStruct((b, m, k), jnp.float32),
        ],
    )(posa_t, posq_t)


def _gather_rows(table, gidx, d):
    rows = gidx.shape[0]
    nw = 32
    ch = 128
    per = rows // nw
    nch = per // ch
    mesh = plsc.VectorSubcoreMesh(core_axis_name="c", subcore_axis_name="s")

    @functools.partial(
        pl.kernel,
        mesh=mesh,
        compiler_params=pltpu.CompilerParams(use_tc_tiling_on_sc=False),
        out_type=jax.ShapeDtypeStruct((rows, d), jnp.float32),
        scratch_types=[
            pltpu.VMEM((ch,), jnp.int32),
            pltpu.VMEM((ch, d), jnp.float32),
            pltpu.SemaphoreType.DMA,
        ],
    )
    def gk(table_hbm, idx_hbm, out_hbm, idx_v, rows_v, sem):
        wid = lax.axis_index("s") * 2 + lax.axis_index("c")
        base = wid * per

        def body(i, carry):
            off = base + i * ch
            pltpu.sync_copy(idx_hbm.at[pl.ds(off, ch)], idx_v)
            pltpu.async_copy(table_hbm.at[idx_v], rows_v, sem).wait()
            pltpu.sync_copy(rows_v, out_hbm.at[pl.ds(off, ch)])
            return carry

        lax.fori_loop(0, nch, body, 0)

    return gk(table, gidx)


def _group_mlp_body(cx, k, rows_ref, posq_ref, bias_ref,
                    w1a_ref, w1b_ref, b1_ref, g1_ref, be1_ref,
                    w2_ref, b2_ref, g2_ref, be2_ref,
                    w3_ref, b3_ref, out_ref):
    mt = posq_ref.shape[0]
    d = rows_ref.shape[2]
    s = jnp.sqrt(jnp.float32(1.0 + _EPS))
    rows = rows_ref[...].reshape(mt * k, d)
    xj = rows[:, :cx]
    pj = rows[:, cx:cx + 3]
    pq = jnp.broadcast_to(posq_ref[...][:, None, :], (mt, k, 3)).reshape(mt * k, 3)
    dp = pj - pq
    h = (jnp.dot(xj, w1a_ref[...], preferred_element_type=jnp.float32)
         + jnp.dot(dp, w1b_ref[...], preferred_element_type=jnp.float32)
         + b1_ref[...][None, :])
    h = h / s * g1_ref[...][None, :] + be1_ref[...][None, :]
    h = jnp.maximum(h, 0.0)
    h = jnp.dot(h, w2_ref[...], preferred_element_type=jnp.float32) + b2_ref[...][None, :]
    h = h / s * g2_ref[...][None, :] + be2_ref[...][None, :]
    h = jnp.maximum(h, 0.0)
    h = jnp.dot(h, w3_ref[...], preferred_element_type=jnp.float32) + b3_ref[...][None, :]
    cout = h.shape[1]
    h3 = h.reshape(mt, k, cout) + bias_ref[...][:, :, None]
    out_ref[...] = jnp.max(h3, axis=1)


def _group_mlp(rows_g, posq, bias, params, cx, k, mt):
    bm, _, d = rows_g.shape
    (w1, b1, g1, be1), (w2, b2, g2, be2), (w3, b3, _, _) = params
    w1a, w1b = w1[:cx], w1[cx:]
    cout = w3.shape[1]
    grid = (bm // mt,)
    full = lambda *s: pl.BlockSpec(s, lambda i: tuple(0 for _ in s))
    return pl.pallas_call(
        functools.partial(_group_mlp_body, cx, k),
        grid=grid,
        in_specs=[
            pl.BlockSpec((mt, k, d), lambda i: (i, 0, 0)),
            pl.BlockSpec((mt, 3), lambda i: (i, 0)),
            pl.BlockSpec((mt, k), lambda i: (i, 0)),
            full(*w1a.shape), full(*w1b.shape), full(*b1.shape),
            full(*g1.shape), full(*be1.shape),
            full(*w2.shape), full(*b2.shape), full(*g2.shape), full(*be2.shape),
            full(*w3.shape), full(*b3.shape),
        ],
        out_specs=pl.BlockSpec((mt, cout), lambda i: (i, 0)),
        out_shape=jax.ShapeDtypeStruct((bm, cout), jnp.float32),
    )(rows_g, posq, bias, w1a, w1b, b1, g1, be1, w2, b2, g2, be2, w3, b3)


def _head_body(b, w1a_ref, w1b_ref, b1_ref, g1_ref, be1_ref,
               w2_ref, b2_ref, g2_ref, be2_ref, w3_ref, b3_ref,
               hw1_ref, hb1_ref, hw2_ref, hb2_ref, hw3_ref, hb3_ref,
               x_ref, p_ref, out_ref):
    s = jnp.sqrt(jnp.float32(1.0 + _EPS))
    bm = x_ref.shape[0]
    m = bm // b
    h = (jnp.dot(x_ref[...], w1a_ref[...], preferred_element_type=jnp.float32)
         + jnp.dot(p_ref[...], w1b_ref[...], preferred_element_type=jnp.float32)
         + b1_ref[...][None, :])
    h = h / s * g1_ref[...][None, :] + be1_ref[...][None, :]
    h = jnp.maximum(h, 0.0)
    h = jnp.dot(h, w2_ref[...], preferred_element_type=jnp.float32) + b2_ref[...][None, :]
    h = h / s * g2_ref[...][None, :] + be2_ref[...][None, :]
    h = jnp.maximum(h, 0.0)
    h = jnp.dot(h, w3_ref[...], preferred_element_type=jnp.float32) + b3_ref[...][None, :]
    g = jnp.max(h.reshape(b, m, h.shape[1]), axis=1)
    t = jnp.dot(g, hw1_ref[...], preferred_element_type=jnp.float32) + hb1_ref[...][None, :]
    t = jnp.maximum(t, 0.0)
    t = jnp.dot(t, hw2_ref[...], preferred_element_type=jnp.float32) + hb2_ref[...][None, :]
    t = jnp.maximum(t, 0.0)
    z = jnp.dot(t, hw3_ref[...], preferred_element_type=jnp.float32) + hb3_ref[...][None, :]
    zmax = jnp.max(z, axis=1, keepdims=True)
    zs = z - zmax
    out_ref[...] = zs - jnp.log(jnp.sum(jnp.exp(zs), axis=1, keepdims=True))


def _head(x2, p2, sa3_mlp, head_mlp, b):
    (w1, b1, g1, be1), (w2, b2, g2, be2), (w3, b3, _, _) = sa3_mlp
    (hw1, hb1, _, _), (hw2, hb2, _, _), (hw3, hb3, _, _) = head_mlp
    w1a, w1b = w1[:-3], w1[-3:]
    args = (w1a, w1b, b1, g1, be1, w2, b2, g2, be2, w3, b3,
            hw1, hb1, hw2, hb2, hw3, hb3, x2, p2)
    return pl.pallas_call(
        functools.partial(_head_body, b),
        out_shape=jax.ShapeDtypeStruct((b, _NCLS), jnp.float32),
    )(*args)


def _stage(x, pos, params, m, k, r, mt):
    b, n, cx = x.shape
    pos_bt = jnp.transpose(pos, (0, 2, 1))
    pos_t = jnp.transpose(pos, (2, 0, 1))
    posq_t3 = _fps(pos_t, m)
    posq_bt = jnp.transpose(posq_t3, (1, 0, 2))
    idx, bias = _knn(pos_bt, posq_bt, k, r)
    d = ((cx + 3 + 15) // 16) * 16
    table = jnp.concatenate([x, pos], axis=-1).reshape(b * n, cx + 3)
    table = jnp.pad(table, ((0, 0), (0, d - (cx + 3))))
    rows_g = _gather_rows(table, idx.reshape(-1), d).reshape(b * m, k, d)
    posq = jnp.transpose(posq_t3, (1, 2, 0)).reshape(b * m, 3)
    xo = _group_mlp(rows_g, posq, bias.reshape(b * m, k), params, cx, k, mt)
    cout = xo.shape[-1]
    return xo.reshape(b, m, cout), posq.reshape(b, m, 3)


def kernel(x, pos, batch, sa1_mlp, sa2_mlp, sa3_mlp, head_mlp):
    del batch
    xb = x.reshape(_B, _N, _F)
    pb = pos.reshape(_B, _N, 3)
    x1, p1 = _stage(xb, pb, sa1_mlp, _SA1[0], _SA1[1], _SA1[2], mt=128)
    x2, p2 = _stage(x1, p1, sa2_mlp, _SA2[0], _SA2[1], _SA2[2], mt=128)
    m2 = _SA2[0]
    return _head(x2.reshape(_B * m2, -1), p2.reshape(_B * m2, 3),
                 sa3_mlp, head_mlp, _B)


# speedup vs baseline: 13.5031x; 1.3526x over previous
import functools

import jax
import jax.numpy as jnp
import numpy as np
from jax import lax
from jax.experimental import pallas as pl
from jax.experimental.pallas import tpu as pltpu
from jax.experimental.pallas import tpu_sc as plsc

_B, _N, _F = 16, 1024, 3
_NCLS = 40
_SA1 = (512, 32, 0.2)
_SA2 = (128, 32, 0.4)
_EPS = 1e-5
_NEG = float("-inf")


def _fps_body(pos_ref, posq_ref):
    _, b, n = pos_ref.shape
    m = posq_ref.shape[2]
    px = pos_ref[0]
    py = pos_ref[1]
    pz = pos_ref[2]
    iota_n = lax.broadcasted_iota(jnp.int32, (b, n), 1)
    iota_m = lax.broadcasted_iota(jnp.int32, (b, m), 1)

    def gather_last(prev):
        eq = iota_n == prev[:, None]
        lx = jnp.sum(jnp.where(eq, px, 0.0), axis=1)
        ly = jnp.sum(jnp.where(eq, py, 0.0), axis=1)
        lz = jnp.sum(jnp.where(eq, pz, 0.0), axis=1)
        return lx, ly, lz

    def body(i, carry):
        prev, dists, qx, qy, qz = carry
        lx, ly, lz = gather_last(prev)
        hit = iota_m == (i - 1)
        qx = jnp.where(hit, lx[:, None], qx)
        qy = jnp.where(hit, ly[:, None], qy)
        qz = jnp.where(hit, lz[:, None], qz)
        dx = px - lx[:, None]
        dy = py - ly[:, None]
        dz = pz - lz[:, None]
        d = (dx * dx + dy * dy) + dz * dz
        dists = jnp.minimum(dists, d)
        am = jnp.argmax(dists, axis=1).astype(jnp.int32)
        return am, dists, qx, qy, qz

    prev0 = jnp.zeros((b,), jnp.int32)
    d0 = jnp.full((b, n), jnp.inf, jnp.float32)
    q0 = jnp.zeros((b, m), jnp.float32)
    prev, _, qx, qy, qz = lax.fori_loop(1, m, body, (prev0, d0, q0, q0, q0))
    lx, ly, lz = gather_last(prev)
    hit = iota_m == (m - 1)
    posq_ref[0] = jnp.where(hit, lx[:, None], qx)
    posq_ref[1] = jnp.where(hit, ly[:, None], qy)
    posq_ref[2] = jnp.where(hit, lz[:, None], qz)


def _fps(pos_t, m):
    _, b, n = pos_t.shape
    return pl.pallas_call(
        _fps_body,
        out_shape=jax.ShapeDtypeStruct((3, b, m), jnp.float32),
    )(pos_t)


def _knn_body(n, k, r2, g, posa_ref, posq_ref, idx_ref, bias_ref):
    m = posq_ref.shape[2]
    qx = posq_ref[:, 0, :]
    qy = posq_ref[:, 1, :]
    qz = posq_ref[:, 2, :]
    px = posa_ref[:, 0, :]
    py = posa_ref[:, 1, :]
    pz = posa_ref[:, 2, :]
    dx = qx[:, :, None] - px[:, None, :]
    dy = qy[:, :, None] - py[:, None, :]
    dz = qz[:, :, None] - pz[:, None, :]
    d2 = (dx * dx + dy * dy) + dz * dz
    iota_n = lax.broadcasted_iota(jnp.int32, (g, m, n), 2)
    iota_k = lax.broadcasted_iota(jnp.int32, (g, m, k), 2)
    base = (pl.program_id(0) * g
            + lax.broadcasted_iota(jnp.int32, (g, m), 0)) * n
    cnt = jnp.sum((d2 <= r2).astype(jnp.int32), axis=2)
    bias_ref[...] = jnp.where(iota_k < cnt[:, :, None], 0.0, _NEG)

    def body(t, carry):
        d2c, idxc = carry
        am = jnp.argmin(d2c, axis=2).astype(jnp.int32)
        idxc = jnp.where(iota_k == t, (am + base)[:, :, None], idxc)
        d2c = jnp.where(iota_n == am[:, :, None], jnp.inf, d2c)
        return d2c, idxc

    idx0 = jnp.zeros((g, m, k), jnp.int32)
    _, idxc = lax.fori_loop(0, k, body, (d2, idx0))
    idx_ref[...] = idxc


def _knn(posa_t, posq_t, k, r, g):
    b, _, n = posa_t.shape
    m = posq_t.shape[2]
    r2 = float(r) * float(r)
    return pl.pallas_call(
        functools.partial(_knn_body, n, k, r2, g),
        grid=(b // g,),
        in_specs=[
            pl.BlockSpec((g, 3, n), lambda i: (i, 0, 0)),
            pl.BlockSpec((g, 3, m), lambda i: (i, 0, 0)),
        ],
        out_specs=[
            pl.BlockSpec((g, m, k), lambda i: (i, 0, 0)),
            pl.BlockSpec((g, m, k), lambda i: (i, 0, 0)),
        ],
        out_shape=[
            jax.ShapeDtypeStruct((b, m, k), jnp.int32),
            jax.ShapeDtypeStruct((b, m, k), jnp.float32),
        ],
    )(posa_t, posq_t)


def _gather_rows(table, gidx, d):
    rows = gidx.shape[0]
    nw = 32
    ch = 128
    per = rows // nw
    nch = per // ch
    cap = max(1, (200 * 1024) // (ch * d * 4))
    nburst = 1
    while nburst * 2 <= cap and nch % (nburst * 2) == 0:
        nburst *= 2
    big = ch * nburst
    nbig = per // big
    mesh = plsc.VectorSubcoreMesh(core_axis_name="c", subcore_axis_name="s")

    @functools.partial(
        pl.kernel,
        mesh=mesh,
        compiler_params=pltpu.CompilerParams(use_tc_tiling_on_sc=False),
        out_type=jax.ShapeDtypeStruct((rows, d), jnp.float32),
        scratch_types=[
            pltpu.VMEM((per,), jnp.int32),
            pltpu.VMEM((big, d), jnp.float32),
            pltpu.SemaphoreType.DMA,
        ],
    )
    def gk(table_hbm, idx_hbm, out_hbm, idx_v, rows_v, sem):
        wid = lax.axis_index("s") * 2 + lax.axis_index("c")
        base = wid * per
        pltpu.sync_copy(idx_hbm.at[pl.ds(base, per)], idx_v)

        def body(i, carry):
            off = i * big
            handles = [
                pltpu.async_copy(
                    table_hbm.at[idx_v.at[pl.ds(off + j * ch, ch)]],
                    rows_v.at[pl.ds(j * ch, ch)],
                    sem,
                )
                for j in range(nburst)
            ]
            for h in handles:
                h.wait()
            pltpu.sync_copy(rows_v, out_hbm.at[pl.ds(base + off, big)])
            return carry

        lax.fori_loop(0, nbig, body, 0)

    return gk(table, gidx)


def _group_mlp_body(cx, k, rows_ref, posq_ref, bias_ref,
                    w1a_ref, w1b_ref, b1_ref, g1_ref, be1_ref,
                    w2_ref, b2_ref, g2_ref, be2_ref,
                    w3_ref, b3_ref, out_ref):
    mt = posq_ref.shape[0]
    d = rows_ref.shape[1]
    s = jnp.sqrt(jnp.float32(1.0 + _EPS))
    rows = rows_ref[...]
    xj = rows[:, :cx]
    pj = rows[:, cx:cx + 3]
    pq = jnp.broadcast_to(posq_ref[...][:, None, :], (mt, k, 3)).reshape(mt * k, 3)
    dp = pj - pq
    h = (jnp.dot(xj, w1a_ref[...], preferred_element_type=jnp.float32)
         + jnp.dot(dp, w1b_ref[...], preferred_element_type=jnp.float32)
         + b1_ref[...][None, :])
    h = h / s * g1_ref[...][None, :] + be1_ref[...][None, :]
    h = jnp.maximum(h, 0.0)
    h = jnp.dot(h, w2_ref[...], preferred_element_type=jnp.float32) + b2_ref[...][None, :]
    h = h / s * g2_ref[...][None, :] + be2_ref[...][None, :]
    h = jnp.maximum(h, 0.0)
    h = jnp.dot(h, w3_ref[...], preferred_element_type=jnp.float32) + b3_ref[...][None, :]
    cout = h.shape[1]
    h3 = h.reshape(mt, k, cout) + bias_ref[...][:, :, None]
    out_ref[...] = jnp.max(h3, axis=1)


def _group_mlp(rows_g, posq, bias, params, cx, k, mt):
    bmk, d = rows_g.shape
    bm = bmk // k
    (w1, b1, g1, be1), (w2, b2, g2, be2), (w3, b3, _, _) = params
    w1a, w1b = w1[:cx], w1[cx:]
    cout = w3.shape[1]
    grid = (bm // mt,)
    full = lambda *s: pl.BlockSpec(s, lambda i: tuple(0 for _ in s))
    return pl.pallas_call(
        functools.partial(_group_mlp_body, cx, k),
        grid=grid,
        in_specs=[
            pl.BlockSpec((mt * k, d), lambda i: (i, 0)),
            pl.BlockSpec((mt, 3), lambda i: (i, 0)),
            pl.BlockSpec((mt, k), lambda i: (i, 0)),
            full(*w1a.shape), full(*w1b.shape), full(*b1.shape),
            full(*g1.shape), full(*be1.shape),
            full(*w2.shape), full(*b2.shape), full(*g2.shape), full(*be2.shape),
            full(*w3.shape), full(*b3.shape),
        ],
        out_specs=pl.BlockSpec((mt, cout), lambda i: (i, 0)),
        out_shape=jax.ShapeDtypeStruct((bm, cout), jnp.float32),
    )(rows_g, posq, bias, w1a, w1b, b1, g1, be1, w2, b2, g2, be2, w3, b3)


def _head_body(b, w1a_ref, w1b_ref, b1_ref, g1_ref, be1_ref,
               w2_ref, b2_ref, g2_ref, be2_ref, w3_ref, b3_ref,
               hw1_ref, hb1_ref, hw2_ref, hb2_ref, hw3_ref, hb3_ref,
               x_ref, p_ref, out_ref):
    s = jnp.sqrt(jnp.float32(1.0 + _EPS))
    bm = x_ref.shape[0]
    m = bm // b
    h = (jnp.dot(x_ref[...], w1a_ref[...], preferred_element_type=jnp.float32)
         + jnp.dot(p_ref[...], w1b_ref[...], preferred_element_type=jnp.float32)
         + b1_ref[...][None, :])
    h = h / s * g1_ref[...][None, :] + be1_ref[...][None, :]
    h = jnp.maximum(h, 0.0)
    h = jnp.dot(h, w2_ref[...], preferred_element_type=jnp.float32) + b2_ref[...][None, :]
    h = h / s * g2_ref[...][None, :] + be2_ref[...][None, :]
    h = jnp.maximum(h, 0.0)
    h = jnp.dot(h, w3_ref[...], preferred_element_type=jnp.float32) + b3_ref[...][None, :]
    g = jnp.max(h.reshape(b, m, h.shape[1]), axis=1)
    t = jnp.dot(g, hw1_ref[...], preferred_element_type=jnp.float32) + hb1_ref[...][None, :]
    t = jnp.maximum(t, 0.0)
    t = jnp.dot(t, hw2_ref[...], preferred_element_type=jnp.float32) + hb2_ref[...][None, :]
    t = jnp.maximum(t, 0.0)
    z = jnp.dot(t, hw3_ref[...], preferred_element_type=jnp.float32) + hb3_ref[...][None, :]
    zmax = jnp.max(z, axis=1, keepdims=True)
    zs = z - zmax
    out_ref[...] = zs - jnp.log(jnp.sum(jnp.exp(zs), axis=1, keepdims=True))


def _head(x2, p2, sa3_mlp, head_mlp, b):
    (w1, b1, g1, be1), (w2, b2, g2, be2), (w3, b3, _, _) = sa3_mlp
    (hw1, hb1, _, _), (hw2, hb2, _, _), (hw3, hb3, _, _) = head_mlp
    w1a, w1b = w1[:-3], w1[-3:]
    args = (w1a, w1b, b1, g1, be1, w2, b2, g2, be2, w3, b3,
            hw1, hb1, hw2, hb2, hw3, hb3, x2, p2)
    return pl.pallas_call(
        functools.partial(_head_body, b),
        out_shape=jax.ShapeDtypeStruct((b, _NCLS), jnp.float32),
    )(*args)


def _stage(x, pos, params, m, k, r, mt, kg):
    b, n, cx = x.shape
    pos_bt = jnp.transpose(pos, (0, 2, 1))
    pos_t = jnp.transpose(pos, (2, 0, 1))
    posq_t3 = _fps(pos_t, m)
    posq_bt = jnp.transpose(posq_t3, (1, 0, 2))
    idx, bias = _knn(pos_bt, posq_bt, k, r, kg)
    d = ((cx + 3 + 15) // 16) * 16
    table = jnp.concatenate([x, pos], axis=-1).reshape(b * n, cx + 3)
    table = jnp.pad(table, ((0, 0), (0, d - (cx + 3))))
    rows_g = _gather_rows(table, idx.reshape(-1), d)
    posq = jnp.transpose(posq_t3, (1, 2, 0)).reshape(b * m, 3)
    xo = _group_mlp(rows_g, posq, bias.reshape(b * m, k), params, cx, k, mt)
    cout = xo.shape[-1]
    return xo.reshape(b, m, cout), posq.reshape(b, m, 3)


def kernel(x, pos, batch, sa1_mlp, sa2_mlp, sa3_mlp, head_mlp):
    del batch
    xb = x.reshape(_B, _N, _F)
    pb = pos.reshape(_B, _N, 3)
    x1, p1 = _stage(xb, pb, sa1_mlp, _SA1[0], _SA1[1], _SA1[2], mt=128, kg=1)
    x2, p2 = _stage(x1, p1, sa2_mlp, _SA2[0], _SA2[1], _SA2[2], mt=128, kg=4)
    m2 = _SA2[0]
    return _head(x2.reshape(_B * m2, -1), p2.reshape(_B * m2, 3),
                 sa3_mlp, head_mlp, _B)
